# baseline (speedup 1.0000x reference)
import jax
import jax.numpy as jnp
from jax import lax
from jax.experimental import pallas as pl
from jax.experimental.pallas import tpu as pltpu
from jax.experimental.pallas import tpu_sc as plsc

_NW = 32
_RG = 4
_COLS = 8192


def _sc_body(x_hbm, m_hbm, o_hbm, xv, mv, ov):
    c = lax.axis_index("c")
    s = lax.axis_index("s")
    wid = s * 2 + c
    rows = x_hbm.shape[0] // _COLS
    rows_per_w = rows // _NW
    base = wid * rows_per_w
    n_groups = rows_per_w // _RG
    nv = _COLS // 16

    def group_fn(g, _):
        e0 = (base + g * _RG) * _COLS
        pltpu.sync_copy(x_hbm.at[pl.ds(e0, _RG * _COLS)], xv)
        pltpu.sync_copy(m_hbm.at[pl.ds(e0, _RG * _COLS)], mv)

        def row_fn(r, _):
            def v_fn(v, carry):
                o = r * _COLS + 16 * v
                z = xv[pl.ds(o, 16)] * mv[pl.ds(o, 16)].astype(jnp.float32)
                cc = plsc.cumsum(z) + carry
                ov[pl.ds(o, 16)] = cc
                return carry + jnp.sum(z)

            lax.fori_loop(0, nv, v_fn, jnp.float32(0.0))
            return 0

        lax.fori_loop(0, _RG, row_fn, 0)
        pltpu.sync_copy(ov, o_hbm.at[pl.ds(e0, _RG * _COLS)])
        return 0

    lax.fori_loop(0, n_groups, group_fn, 0)


def kernel(x, mask):
    rows, cols = x.shape
    m32 = mask.astype(jnp.int32).reshape(rows * cols)
    mesh = plsc.VectorSubcoreMesh(core_axis_name="c", subcore_axis_name="s")
    f = pl.kernel(
        _sc_body,
        out_type=jax.ShapeDtypeStruct((rows * cols,), jnp.float32),
        mesh=mesh,
        scratch_types=[
            pltpu.VMEM((_RG * cols,), jnp.float32),
            pltpu.VMEM((_RG * cols,), jnp.int32),
            pltpu.VMEM((_RG * cols,), jnp.float32),
        ],
        compiler_params=pltpu.CompilerParams(needs_layout_passes=False),
    )
    return f(x.reshape(rows * cols), m32).reshape(rows, cols)


# speedup vs baseline: 1.9669x; 1.9669x over previous
import jax
import jax.numpy as jnp
from jax import lax
from jax.experimental import pallas as pl
from jax.experimental.pallas import tpu as pltpu
from jax.experimental.pallas import tpu_sc as plsc

_NW = 32
_RG = 4
_COLS = 8192

_GATHER_DNUMS = lax.GatherDimensionNumbers(
    offset_dims=(), collapsed_slice_dims=(0,), start_index_map=(0,)
)


def _bcast_last(v):
    idx = jnp.full((16, 1), 15, jnp.int32)
    return lax.gather(
        v, idx, _GATHER_DNUMS, (1,),
        mode=lax.GatherScatterMode.PROMISE_IN_BOUNDS,
    )


def _sc_body(x_hbm, m_hbm, o_hbm, xv, mv, ov):
    c = lax.axis_index("c")
    s = lax.axis_index("s")
    wid = s * 2 + c
    rows = x_hbm.shape[0] // _COLS
    rows_per_w = rows // _NW
    base = wid * rows_per_w
    n_groups = rows_per_w // _RG
    nv = _COLS // 16

    def group_fn(g, _):
        e0 = (base + g * _RG) * _COLS
        pltpu.sync_copy(x_hbm.at[pl.ds(e0, _RG * _COLS)], xv)
        pltpu.sync_copy(m_hbm.at[pl.ds(e0, _RG * _COLS)], mv)

        def row_fn(r, _):
            def t_fn(t, carry):
                o = r * _COLS + 128 * t
                zs = [
                    xv[pl.ds(o + 16 * j, 16)]
                    * mv[pl.ds(o + 16 * j, 16)].astype(jnp.float32)
                    for j in range(8)
                ]
                cs = [plsc.cumsum(z) for z in zs]
                ts = [_bcast_last(cv) for cv in cs]
                acc = carry
                for j in range(8):
                    ov[pl.ds(o + 16 * j, 16)] = cs[j] + acc
                    acc = acc + ts[j]
                return acc

            lax.fori_loop(0, nv // 8, t_fn, jnp.zeros((16,), jnp.float32))
            return 0

        lax.fori_loop(0, _RG, row_fn, 0)
        pltpu.sync_copy(ov, o_hbm.at[pl.ds(e0, _RG * _COLS)])
        return 0

    lax.fori_loop(0, n_groups, group_fn, 0)


def kernel(x, mask):
    rows, cols = x.shape
    m32 = mask.astype(jnp.int32).reshape(rows * cols)
    mesh = plsc.VectorSubcoreMesh(core_axis_name="c", subcore_axis_name="s")
    f = pl.kernel(
        _sc_body,
        out_type=jax.ShapeDtypeStruct((rows * cols,), jnp.float32),
        mesh=mesh,
        scratch_types=[
            pltpu.VMEM((_RG * cols,), jnp.float32),
            pltpu.VMEM((_RG * cols,), jnp.int32),
            pltpu.VMEM((_RG * cols,), jnp.float32),
        ],
        compiler_params=pltpu.CompilerParams(needs_layout_passes=False),
    )
    return f(x.reshape(rows * cols), m32).reshape(rows, cols)


# speedup vs baseline: 3.6472x; 1.8543x over previous
import jax
import jax.numpy as jnp
from jax import lax
from jax.experimental import pallas as pl
from jax.experimental.pallas import tpu as pltpu
from jax.experimental.pallas import tpu_sc as plsc


_BR = 512
_BC = 2048
_G = 256


def _scan_small(a):
    w = a.shape[1]
    k = 1
    while k < w:
        a = a + jnp.concatenate(
            [jnp.zeros((a.shape[0], k), a.dtype), a[:, :-k]], axis=1
        )
        k *= 2
    return a


def _tc_body(x_ref, m_ref, o_ref, acc_ref):
    j = pl.program_id(1)

    @pl.when(j == 0)
    def _():
        acc_ref[...] = jnp.zeros_like(acc_ref)

    masked = x_ref[...] * m_ref[...].astype(jnp.float32)
    ng = _BC // _G
    row = lax.broadcasted_iota(jnp.int32, (_G, _G), 0)
    col = lax.broadcasted_iota(jnp.int32, (_G, _G), 1)
    tri = (row <= col).astype(jnp.float32)

    local = [
        jnp.dot(masked[:, g * _G:(g + 1) * _G], tri,
                preferred_element_type=jnp.float32)
        for g in range(ng)
    ]
    gs = jnp.concatenate([l[:, _G - 1:_G] for l in local], axis=1)
    incl = _scan_small(gs)
    offs = incl - gs + acc_ref[...][:, :1]

    for g in range(ng):
        o_ref[:, g * _G:(g + 1) * _G] = local[g] + offs[:, g:g + 1]

    total = offs[:, ng - 1:ng] + gs[:, ng - 1:ng]
    acc_ref[...] = jnp.broadcast_to(total, acc_ref.shape)


def _tc_cumsum(x, mask):
    rows, cols = x.shape
    grid = (rows // _BR, cols // _BC)
    return pl.pallas_call(
        _tc_body,
        grid=grid,
        in_specs=[
            pl.BlockSpec((_BR, _BC), lambda i, j: (i, j)),
            pl.BlockSpec((_BR, _BC), lambda i, j: (i, j)),
        ],
        out_specs=pl.BlockSpec((_BR, _BC), lambda i, j: (i, j)),
        out_shape=jax.ShapeDtypeStruct((rows, cols), jnp.float32),
        scratch_shapes=[pltpu.VMEM((_BR, 128), jnp.float32)],
        compiler_params=pltpu.CompilerParams(
            dimension_semantics=("parallel", "arbitrary"),
        ),
    )(x, mask)


_NW = 32
_RG = 4
_COLS = 8192

_GATHER_DNUMS = lax.GatherDimensionNumbers(
    offset_dims=(), collapsed_slice_dims=(0,), start_index_map=(0,)
)


def _bcast_last(v):
    idx = jnp.full((16, 1), 15, jnp.int32)
    return lax.gather(
        v, idx, _GATHER_DNUMS, (1,),
        mode=lax.GatherScatterMode.PROMISE_IN_BOUNDS,
    )


def _sc_body(x_hbm, m_hbm, o_hbm, xv, mv, ov):
    c = lax.axis_index("c")
    s = lax.axis_index("s")
    wid = s * 2 + c
    rows = x_hbm.shape[0] // _COLS
    rows_per_w = rows // _NW
    base = wid * rows_per_w
    n_groups = rows_per_w // _RG
    nv = _COLS // 16

    def group_fn(g, _):
        e0 = (base + g * _RG) * _COLS
        pltpu.sync_copy(x_hbm.at[pl.ds(e0, _RG * _COLS)], xv)
        pltpu.sync_copy(m_hbm.at[pl.ds(e0, _RG * _COLS)], mv)

        def row_fn(r, _):
            def t_fn(t, carry):
                o = r * _COLS + 128 * t
                zs = [
                    xv[pl.ds(o + 16 * j, 16)]
                    * mv[pl.ds(o + 16 * j, 16)].astype(jnp.float32)
                    for j in range(8)
                ]
                cs = [plsc.cumsum(z) for z in zs]
                ts = [_bcast_last(cv) for cv in cs]
                acc = carry
                for j in range(8):
                    ov[pl.ds(o + 16 * j, 16)] = cs[j] + acc
                    acc = acc + ts[j]
                return acc

            lax.fori_loop(0, nv // 8, t_fn, jnp.zeros((16,), jnp.float32))
            return 0

        lax.fori_loop(0, _RG, row_fn, 0)
        pltpu.sync_copy(ov, o_hbm.at[pl.ds(e0, _RG * _COLS)])
        return 0

    lax.fori_loop(0, n_groups, group_fn, 0)


def _sc_cumsum(x, mask):
    rows, cols = x.shape
    m32 = mask.astype(jnp.int32).reshape(rows * cols)
    mesh = plsc.VectorSubcoreMesh(core_axis_name="c", subcore_axis_name="s")
    f = pl.kernel(
        _sc_body,
        out_type=jax.ShapeDtypeStruct((rows * cols,), jnp.float32),
        mesh=mesh,
        scratch_types=[
            pltpu.VMEM((_RG * cols,), jnp.float32),
            pltpu.VMEM((_RG * cols,), jnp.int32),
            pltpu.VMEM((_RG * cols,), jnp.float32),
        ],
        compiler_params=pltpu.CompilerParams(needs_layout_passes=False),
    )
    return f(x.reshape(rows * cols), m32).reshape(rows, cols)


_SC_ROWS = 512


def kernel(x, mask):
    rows, _ = x.shape
    nt = rows - _SC_ROWS
    out_tc = _tc_cumsum(x[:nt], mask[:nt])
    out_sc = _sc_cumsum(x[nt:], mask[nt:])
    return jnp.concatenate([out_tc, out_sc], axis=0)


# speedup vs baseline: 8.5128x; 2.3341x over previous
import jax
import jax.numpy as jnp
from jax.experimental import pallas as pl
from jax.experimental.pallas import tpu as pltpu

_BR = 1024
_BC = 2048
_G = 256


def _scan_small(a):
    w = a.shape[1]
    k = 1
    while k < w:
        a = a + jnp.concatenate(
            [jnp.zeros((a.shape[0], k), a.dtype), a[:, :-k]], axis=1
        )
        k *= 2
    return a


def _body(x_ref, m_ref, o_ref, acc_ref):
    j = pl.program_id(1)

    @pl.when(j == 0)
    def _():
        acc_ref[...] = jnp.zeros_like(acc_ref)

    masked = x_ref[...] * m_ref[...].astype(jnp.float32)
    ng = _BC // _G
    row = jax.lax.broadcasted_iota(jnp.int32, (_G, _G), 0)
    col = jax.lax.broadcasted_iota(jnp.int32, (_G, _G), 1)
    tri = (row <= col).astype(jnp.float32)

    local = [
        jnp.dot(masked[:, g * _G:(g + 1) * _G], tri,
                preferred_element_type=jnp.float32)
        for g in range(ng)
    ]
    gs = jnp.concatenate([l[:, _G - 1:_G] for l in local], axis=1)
    incl = _scan_small(gs)
    offs = incl - gs + acc_ref[...][:, :1]

    for g in range(ng):
        o_ref[:, g * _G:(g + 1) * _G] = local[g] + offs[:, g:g + 1]

    total = offs[:, ng - 1:ng] + gs[:, ng - 1:ng]
    acc_ref[...] = jnp.broadcast_to(total, acc_ref.shape)


def kernel(x, mask):
    rows, cols = x.shape
    grid = (rows // _BR, cols // _BC)
    return pl.pallas_call(
        _body,
        grid=grid,
        in_specs=[
            pl.BlockSpec((_BR, _BC), lambda i, j: (i, j)),
            pl.BlockSpec((_BR, _BC), lambda i, j: (i, j)),
        ],
        out_specs=pl.BlockSpec((_BR, _BC), lambda i, j: (i, j)),
        out_shape=jax.ShapeDtypeStruct((rows, cols), jnp.float32),
        scratch_shapes=[pltpu.VMEM((_BR, 128), jnp.float32)],
        compiler_params=pltpu.CompilerParams(
            dimension_semantics=("parallel", "arbitrary"),
        ),
    )(x, mask)
